# Initial kernel scaffold; baseline (speedup 1.0000x reference)
import jax
import jax.numpy as jnp
from jax.experimental import pallas as pl


def kernel(x, edge_index, edge_f, edge_attr, W1, b1, W2, b2):
    raise NotImplementedError("write your pallas kernel here")


# speedup vs baseline: 102.8127x; 102.8127x over previous
import functools

import jax
import jax.numpy as jnp
from jax import lax
from jax.experimental import pallas as pl
from jax.experimental.pallas import tpu as pltpu
from jax.experimental.pallas import tpu_sc as plsc

N = 100000
E = 6400000
NPAD = 100352
F = 8
NC, NS = 2, 16
NW = NC * NS
RPT = NPAD // NS
V8 = E // 1024
V8PW = V8 // NW
K8 = 3
NCHUNK = V8PW // K8
REM8 = NW * V8PW

_mesh = plsc.VectorSubcoreMesh(core_axis_name="c", subcore_axis_name="s")


def _deg_pass(dst2d, zeros1):
    @functools.partial(
        pl.kernel,
        mesh=_mesh,
        compiler_params=pltpu.CompilerParams(use_tc_tiling_on_sc=False),
        out_type=jax.ShapeDtypeStruct((NC, 1, NPAD), jnp.float32),
        scratch_types=[
            pltpu.VMEM((K8, 8, 128), jnp.int32),
            pltpu.VMEM((128,), jnp.float32),
            pltpu.VMEM_SHARED((NPAD,), jnp.float32),
        ],
    )
    def kfn(dst_hbm, z_hbm, out_hbm, dst_v, ones_v, deg_sh):
        c = lax.axis_index("c")
        s = lax.axis_index("s")
        wid = s * NC + c
        for i in range(8):
            ones_v[pl.ds(i * 16, 16)] = jnp.ones((16,), jnp.float32)
        r0 = s * RPT
        pltpu.sync_copy(z_hbm.at[pl.ds(r0, RPT)], deg_sh.at[pl.ds(r0, RPT)])
        plsc.subcore_barrier()
        vbase = wid * V8PW

        def chunk(i, carry):
            pltpu.sync_copy(dst_hbm.at[pl.ds(vbase + i * K8, K8)], dst_v)

            def vec(j8, carry2):
                for r in range(8):
                    pltpu.sync_copy(ones_v, deg_sh.at[dst_v.at[j8, r]],
                                    add=True)
                return carry2

            return lax.fori_loop(0, K8, vec, carry)

        lax.fori_loop(0, NCHUNK, chunk, 0)

        @pl.when(wid < V8 - REM8)
        def _():
            pltpu.sync_copy(dst_hbm.at[pl.ds(REM8 + wid, 1)],
                            dst_v.at[pl.ds(0, 1)])
            for r in range(8):
                pltpu.sync_copy(ones_v, deg_sh.at[dst_v.at[0, r]], add=True)

        plsc.subcore_barrier()
        pltpu.sync_copy(deg_sh.at[pl.ds(r0, RPT)],
                        out_hbm.at[c, 0, pl.ds(r0, RPT)])

    return kfn(dst2d, zeros1)


def _edge_pass(src2d, dst2d, g_rows, zeros8):
    @functools.partial(
        pl.kernel,
        mesh=_mesh,
        compiler_params=pltpu.CompilerParams(use_tc_tiling_on_sc=False),
        out_type=jax.ShapeDtypeStruct((NC, NPAD, F), jnp.float32),
        scratch_types=[
            pltpu.VMEM((K8, 8, 128), jnp.int32),
            pltpu.VMEM((K8, 8, 128), jnp.int32),
            pltpu.VMEM((K8 * 8, 128, F), jnp.float32),
            pltpu.VMEM_SHARED((NPAD, F), jnp.float32),
            pltpu.VMEM_SHARED((NPAD, F), jnp.float32),
        ],
    )
    def kfn(src_hbm, dst_hbm, g_hbm, z_hbm, out_hbm,
            src_v, dst_v, rows_v, g_sh, s_sh):
        c = lax.axis_index("c")
        s = lax.axis_index("s")
        wid = s * NC + c
        r0 = s * RPT
        pltpu.sync_copy(g_hbm.at[pl.ds(r0, RPT)], g_sh.at[pl.ds(r0, RPT)])
        pltpu.sync_copy(z_hbm.at[pl.ds(r0, RPT)], s_sh.at[pl.ds(r0, RPT)])
        plsc.subcore_barrier()
        vbase = wid * V8PW

        def chunk(i, carry):
            vb = vbase + i * K8
            pltpu.sync_copy(src_hbm.at[pl.ds(vb, K8)], src_v)
            pltpu.sync_copy(dst_hbm.at[pl.ds(vb, K8)], dst_v)

            def vec(j8, carry2):
                for r in range(8):
                    j = j8 * 8 + r
                    pltpu.sync_copy(g_sh.at[src_v.at[j8, r]], rows_v.at[j])
                    pltpu.sync_copy(rows_v.at[j], s_sh.at[dst_v.at[j8, r]],
                                    add=True)
                return carry2

            return lax.fori_loop(0, K8, vec, carry)

        lax.fori_loop(0, NCHUNK, chunk, 0)

        @pl.when(wid < V8 - REM8)
        def _():
            pltpu.sync_copy(src_hbm.at[pl.ds(REM8 + wid, 1)],
                            src_v.at[pl.ds(0, 1)])
            pltpu.sync_copy(dst_hbm.at[pl.ds(REM8 + wid, 1)],
                            dst_v.at[pl.ds(0, 1)])
            for r in range(8):
                pltpu.sync_copy(g_sh.at[src_v.at[0, r]], rows_v.at[r])
                pltpu.sync_copy(rows_v.at[r], s_sh.at[dst_v.at[0, r]],
                                add=True)

        plsc.subcore_barrier()
        pltpu.sync_copy(s_sh.at[pl.ds(r0, RPT)], out_hbm.at[c, pl.ds(r0, RPT)])

    return kfn(src2d, dst2d, g_rows, zeros8)


_HI = jax.lax.Precision.HIGHEST


def _tc1_body(dp_ref, xt_ref, w1t_ref, dinv_ref, g1t_ref):
    dp = dp_ref[...]
    deg = dp[0] + dp[1] + 1.0
    dinv = lax.rsqrt(deg)
    h = jnp.dot(w1t_ref[...], xt_ref[...], precision=_HI,
                preferred_element_type=jnp.float32)
    dinv_ref[...] = dinv
    g1t_ref[...] = h * dinv


def _tc2_body(sp_ref, g1t_ref, dinv_ref, b1_ref, w2tp_ref, g2t_ref):
    a = sp_ref[...]
    dinv = dinv_ref[...]
    out1 = dinv * (a[0] + a[1] + g1t_ref[...]) + b1_ref[...]
    h2 = jnp.dot(w2tp_ref[...], out1, precision=_HI,
                 preferred_element_type=jnp.float32)
    g2t_ref[...] = h2 * dinv


def _tc3_body(sp_ref, g2t_ref, dinv_ref, b2_ref, out_ref):
    a = sp_ref[...]
    out_ref[...] = dinv_ref[...] * (a[0] + a[1] + g2t_ref[...]) + b2_ref[...]


def kernel(x, edge_index, edge_f, edge_attr, W1, b1, W2, b2):
    src2d = edge_index[0].reshape(V8, 8, 128)
    dst2d = edge_index[1].reshape(V8, 8, 128)
    zeros1 = jnp.zeros((NPAD,), jnp.float32)
    zeros8 = jnp.zeros((NPAD, F), jnp.float32)
    xt = jnp.pad(x.T, ((0, 0), (0, NPAD - N)))
    w1t = W1.T
    w2tp = jnp.zeros((8, 8), jnp.float32).at[:5].set(W2.T)
    b1c = b1.reshape(8, 1)
    b2c = jnp.zeros((8,), jnp.float32).at[:5].set(b2).reshape(8, 1)

    dp = _deg_pass(dst2d, zeros1)

    dinv_t, g1t = pl.pallas_call(
        _tc1_body,
        out_shape=[jax.ShapeDtypeStruct((1, NPAD), jnp.float32),
                   jax.ShapeDtypeStruct((8, NPAD), jnp.float32)],
    )(dp, xt, w1t)

    s1p = _edge_pass(src2d, dst2d, g1t.T, zeros8)

    g2t = pl.pallas_call(
        _tc2_body,
        out_shape=jax.ShapeDtypeStruct((8, NPAD), jnp.float32),
    )(jnp.transpose(s1p, (0, 2, 1)), g1t, dinv_t, b1c, w2tp)

    s2p = _edge_pass(src2d, dst2d, g2t.T, zeros8)

    outt = pl.pallas_call(
        _tc3_body,
        out_shape=jax.ShapeDtypeStruct((8, NPAD), jnp.float32),
    )(jnp.transpose(s2p, (0, 2, 1)), g2t, dinv_t, b2c)

    return outt.T[:N, :5]


# speedup vs baseline: 167.1200x; 1.6255x over previous
import functools

import jax
import jax.numpy as jnp
from jax import lax
from jax.experimental import pallas as pl
from jax.experimental.pallas import tpu as pltpu
from jax.experimental.pallas import tpu_sc as plsc

N = 100000
E = 6400000
NPAD = 100352
F = 8
NC, NS = 2, 16
NW = NC * NS
RPT = NPAD // NS
V8 = E // 1024
V8PW = V8 // NW
K8 = 5
NCHUNK = V8PW // K8
REM8 = NW * V8PW

_mesh = plsc.VectorSubcoreMesh(core_axis_name="c", subcore_axis_name="s")


def _deg_pass(dst2d, zeros1):
    @functools.partial(
        pl.kernel,
        mesh=_mesh,
        compiler_params=pltpu.CompilerParams(use_tc_tiling_on_sc=False),
        out_type=jax.ShapeDtypeStruct((NC, 1, NPAD), jnp.float32),
        scratch_types=[
            pltpu.VMEM((K8, 8, 128), jnp.int32),
            pltpu.VMEM((128,), jnp.float32),
            pltpu.VMEM_SHARED((NPAD,), jnp.float32),
            pltpu.SemaphoreType.DMA,
        ],
    )
    def kfn(dst_hbm, z_hbm, out_hbm, dst_v, ones_v, deg_sh, ssem):
        c = lax.axis_index("c")
        s = lax.axis_index("s")
        wid = s * NC + c
        for i in range(8):
            ones_v[pl.ds(i * 16, 16)] = jnp.ones((16,), jnp.float32)
        r0 = s * RPT
        pltpu.sync_copy(z_hbm.at[pl.ds(r0, RPT)], deg_sh.at[pl.ds(r0, RPT)])
        plsc.subcore_barrier()
        vbase = wid * V8PW

        def chunk(i, carry):
            pltpu.sync_copy(dst_hbm.at[pl.ds(vbase + i * K8, K8)], dst_v)

            def vec(j8, carry2):
                for r in range(8):
                    pltpu.async_copy(ones_v, deg_sh.at[dst_v.at[j8, r]], ssem,
                                     add=True)

                @pl.when(j8 > 0)
                def _():
                    for r in range(8):
                        pltpu.make_async_copy(
                            ones_v, deg_sh.at[dst_v.at[j8 - 1, r]], ssem
                        ).wait()

                return carry2

            lax.fori_loop(0, K8, vec, 0)
            for r in range(8):
                pltpu.make_async_copy(
                    ones_v, deg_sh.at[dst_v.at[K8 - 1, r]], ssem).wait()
            return carry

        lax.fori_loop(0, NCHUNK, chunk, 0)

        @pl.when(wid < V8 - REM8)
        def _():
            pltpu.sync_copy(dst_hbm.at[pl.ds(REM8 + wid, 1)],
                            dst_v.at[pl.ds(0, 1)])
            for r in range(8):
                pltpu.sync_copy(ones_v, deg_sh.at[dst_v.at[0, r]], add=True)

        plsc.subcore_barrier()
        pltpu.sync_copy(deg_sh.at[pl.ds(r0, RPT)],
                        out_hbm.at[c, 0, pl.ds(r0, RPT)])

    return kfn(dst2d, zeros1)


def _edge_pass(src2d, dst2d, g_rows, zeros8):
    @functools.partial(
        pl.kernel,
        mesh=_mesh,
        compiler_params=pltpu.CompilerParams(use_tc_tiling_on_sc=False),
        out_type=jax.ShapeDtypeStruct((NC, NPAD, F), jnp.float32),
        scratch_types=[
            pltpu.VMEM((K8, 8, 128), jnp.int32),
            pltpu.VMEM((K8, 8, 128), jnp.int32),
            pltpu.VMEM((2, 8, 128, F), jnp.float32),
            pltpu.VMEM_SHARED((NPAD, F), jnp.float32),
            pltpu.VMEM_SHARED((NPAD, F), jnp.float32),
            pltpu.SemaphoreType.DMA,
            pltpu.SemaphoreType.DMA,
        ],
    )
    def kfn(src_hbm, dst_hbm, g_hbm, z_hbm, out_hbm,
            src_v, dst_v, rows_v, g_sh, s_sh, gsem, ssem):
        c = lax.axis_index("c")
        s = lax.axis_index("s")
        wid = s * NC + c
        r0 = s * RPT
        pltpu.sync_copy(g_hbm.at[pl.ds(r0, RPT)], g_sh.at[pl.ds(r0, RPT)])
        pltpu.sync_copy(z_hbm.at[pl.ds(r0, RPT)], s_sh.at[pl.ds(r0, RPT)])
        plsc.subcore_barrier()
        vbase = wid * V8PW

        def chunk(i, carry):
            vb = vbase + i * K8
            pltpu.sync_copy(src_hbm.at[pl.ds(vb, K8)], src_v)
            pltpu.sync_copy(dst_hbm.at[pl.ds(vb, K8)], dst_v)

            def vec(j8, carry2):
                p = lax.rem(j8, 2)
                for r in range(8):
                    pltpu.async_copy(g_sh.at[src_v.at[j8, r]],
                                     rows_v.at[p, r], gsem)

                @pl.when(j8 > 0)
                def _():
                    for r in range(8):
                        pltpu.make_async_copy(
                            rows_v.at[1 - p, r],
                            s_sh.at[dst_v.at[j8 - 1, r]], ssem).wait()

                for r in range(8):
                    pltpu.make_async_copy(g_sh.at[src_v.at[j8, r]],
                                          rows_v.at[p, r], gsem).wait()
                    pltpu.async_copy(rows_v.at[p, r],
                                     s_sh.at[dst_v.at[j8, r]], ssem, add=True)
                return carry2

            lax.fori_loop(0, K8, vec, 0)
            for r in range(8):
                pltpu.make_async_copy(rows_v.at[(K8 - 1) % 2, r],
                                      s_sh.at[dst_v.at[K8 - 1, r]],
                                      ssem).wait()
            return carry

        lax.fori_loop(0, NCHUNK, chunk, 0)

        @pl.when(wid < V8 - REM8)
        def _():
            pltpu.sync_copy(src_hbm.at[pl.ds(REM8 + wid, 1)],
                            src_v.at[pl.ds(0, 1)])
            pltpu.sync_copy(dst_hbm.at[pl.ds(REM8 + wid, 1)],
                            dst_v.at[pl.ds(0, 1)])
            for r in range(8):
                pltpu.sync_copy(g_sh.at[src_v.at[0, r]], rows_v.at[0, r])
                pltpu.sync_copy(rows_v.at[0, r], s_sh.at[dst_v.at[0, r]],
                                add=True)

        plsc.subcore_barrier()
        pltpu.sync_copy(s_sh.at[pl.ds(r0, RPT)], out_hbm.at[c, pl.ds(r0, RPT)])

    return kfn(src2d, dst2d, g_rows, zeros8)


_HI = jax.lax.Precision.HIGHEST


def _tc1_body(dp_ref, xt_ref, w1t_ref, dinv_ref, g1t_ref):
    dp = dp_ref[...]
    deg = dp[0] + dp[1] + 1.0
    dinv = lax.rsqrt(deg)
    h = jnp.dot(w1t_ref[...], xt_ref[...], precision=_HI,
                preferred_element_type=jnp.float32)
    dinv_ref[...] = dinv
    g1t_ref[...] = h * dinv


def _tc2_body(sp_ref, g1t_ref, dinv_ref, b1_ref, w2tp_ref, g2t_ref):
    a = sp_ref[...]
    dinv = dinv_ref[...]
    out1 = dinv * (a[0] + a[1] + g1t_ref[...]) + b1_ref[...]
    h2 = jnp.dot(w2tp_ref[...], out1, precision=_HI,
                 preferred_element_type=jnp.float32)
    g2t_ref[...] = h2 * dinv


def _tc3_body(sp_ref, g2t_ref, dinv_ref, b2_ref, out_ref):
    a = sp_ref[...]
    out_ref[...] = dinv_ref[...] * (a[0] + a[1] + g2t_ref[...]) + b2_ref[...]


def kernel(x, edge_index, edge_f, edge_attr, W1, b1, W2, b2):
    src2d = edge_index[0].reshape(V8, 8, 128)
    dst2d = edge_index[1].reshape(V8, 8, 128)
    zeros1 = jnp.zeros((NPAD,), jnp.float32)
    zeros8 = jnp.zeros((NPAD, F), jnp.float32)
    xt = jnp.pad(x.T, ((0, 0), (0, NPAD - N)))
    w1t = W1.T
    w2tp = jnp.zeros((8, 8), jnp.float32).at[:5].set(W2.T)
    b1c = b1.reshape(8, 1)
    b2c = jnp.zeros((8,), jnp.float32).at[:5].set(b2).reshape(8, 1)

    dp = _deg_pass(dst2d, zeros1)

    dinv_t, g1t = pl.pallas_call(
        _tc1_body,
        out_shape=[jax.ShapeDtypeStruct((1, NPAD), jnp.float32),
                   jax.ShapeDtypeStruct((8, NPAD), jnp.float32)],
    )(dp, xt, w1t)

    s1p = _edge_pass(src2d, dst2d, g1t.T, zeros8)

    g2t = pl.pallas_call(
        _tc2_body,
        out_shape=jax.ShapeDtypeStruct((8, NPAD), jnp.float32),
    )(jnp.transpose(s1p, (0, 2, 1)), g1t, dinv_t, b1c, w2tp)

    s2p = _edge_pass(src2d, dst2d, g2t.T, zeros8)

    outt = pl.pallas_call(
        _tc3_body,
        out_shape=jax.ShapeDtypeStruct((8, NPAD), jnp.float32),
    )(jnp.transpose(s2p, (0, 2, 1)), g2t, dinv_t, b2c)

    return outt.T[:N, :5]


# speedup vs baseline: 180.6724x; 1.0811x over previous
import functools

import jax
import jax.numpy as jnp
from jax import lax
from jax.experimental import pallas as pl
from jax.experimental.pallas import tpu as pltpu
from jax.experimental.pallas import tpu_sc as plsc

N = 100000
E = 6400000
NPAD = 100352
F = 8
NC, NS = 2, 16
NW = NC * NS
RPT = NPAD // NS
V8 = E // 1024
V8PW = V8 // NW
K8 = 5
NCHUNK = V8PW // K8
REM8 = NW * V8PW
NPT = NPAD // NW
CB = 448
NCB = NPT // CB
CBR = CB * F // 16

_mesh = plsc.VectorSubcoreMesh(core_axis_name="c", subcore_axis_name="s")
_sc_params = pltpu.CompilerParams(use_tc_tiling_on_sc=False,
                                 needs_layout_passes=False)


def _lane_d8():
    iota = lax.iota(jnp.int32, 16)
    return (lax.shift_right_logical(iota, 3),
            lax.bitwise_and(iota, 7))


def _rsqrt16(deg):
    i = plsc.bitcast(deg, jnp.int32)
    i = jnp.int32(0x5F3759DF) - lax.shift_right_arithmetic(i, 1)
    y = plsc.bitcast(i, jnp.float32)
    for _ in range(3):
        y = y * (1.5 - 0.5 * deg * y * y)
    return y


def _deg_pass(dst3d, zeros1):
    @functools.partial(
        pl.kernel,
        mesh=_mesh,
        compiler_params=_sc_params,
        out_type=jax.ShapeDtypeStruct((NC * NPAD,), jnp.float32),
        scratch_types=[
            pltpu.VMEM((K8, 8, 128), jnp.int32),
            pltpu.VMEM((128,), jnp.float32),
            pltpu.VMEM_SHARED((NPAD,), jnp.float32),
            pltpu.SemaphoreType.DMA,
        ],
    )
    def kfn(dst_hbm, z_hbm, out_hbm, dst_v, ones_v, deg_sh, ssem):
        c = lax.axis_index("c")
        s = lax.axis_index("s")
        wid = s * NC + c
        for i in range(8):
            ones_v[pl.ds(i * 16, 16)] = jnp.ones((16,), jnp.float32)
        r0 = s * RPT
        pltpu.sync_copy(z_hbm.at[pl.ds(r0, RPT)], deg_sh.at[pl.ds(r0, RPT)])
        plsc.subcore_barrier()
        vbase = wid * V8PW

        def chunk(i, carry):
            pltpu.sync_copy(dst_hbm.at[pl.ds(vbase + i * K8, K8)], dst_v)

            def vec(j8, carry2):
                for r in range(8):
                    pltpu.async_copy(ones_v, deg_sh.at[dst_v.at[j8, r]], ssem,
                                     add=True)

                @pl.when(j8 > 0)
                def _():
                    for r in range(8):
                        pltpu.make_async_copy(
                            ones_v, deg_sh.at[dst_v.at[j8 - 1, r]], ssem
                        ).wait()

                return carry2

            lax.fori_loop(0, K8, vec, 0)
            for r in range(8):
                pltpu.make_async_copy(
                    ones_v, deg_sh.at[dst_v.at[K8 - 1, r]], ssem).wait()
            return carry

        lax.fori_loop(0, NCHUNK, chunk, 0)

        @pl.when(wid < V8 - REM8)
        def _():
            pltpu.sync_copy(dst_hbm.at[pl.ds(REM8 + wid, 1)],
                            dst_v.at[pl.ds(0, 1)])
            for r in range(8):
                pltpu.sync_copy(ones_v, deg_sh.at[dst_v.at[0, r]], add=True)

        plsc.subcore_barrier()
        pltpu.sync_copy(deg_sh.at[pl.ds(r0, RPT)],
                        out_hbm.at[pl.ds(c * NPAD + r0, RPT)])

    return kfn(dst3d, zeros1)


def _stage1(dp, xt8, w1rep):
    @functools.partial(
        pl.kernel,
        mesh=_mesh,
        compiler_params=_sc_params,
        out_type=[jax.ShapeDtypeStruct((NPAD, F), jnp.float32),
                  jax.ShapeDtypeStruct((NPAD,), jnp.float32)],
        scratch_types=[
            pltpu.VMEM((CB,), jnp.float32),
            pltpu.VMEM((CB,), jnp.float32),
            pltpu.VMEM((5 * CB,), jnp.float32),
            pltpu.VMEM((CB,), jnp.float32),
            pltpu.VMEM((CB, F), jnp.float32),
            pltpu.VMEM((5, 16), jnp.float32),
        ],
    )
    def kfn(dp_hbm, x_hbm, w1_hbm, g_hbm, dinv_hbm,
            dpa_v, dpb_v, xc_v, dinv_v, rows_v, w1_v):
        c = lax.axis_index("c")
        s = lax.axis_index("s")
        wid = s * NC + c
        n0 = wid * NPT
        pltpu.sync_copy(w1_hbm, w1_v)
        d8, i7 = _lane_d8()
        w1vec = [w1_v[k, :] for k in range(5)]

        def chunk(ci, carry):
            c0 = n0 + ci * CB
            pltpu.sync_copy(dp_hbm.at[pl.ds(c0, CB)], dpa_v)
            pltpu.sync_copy(dp_hbm.at[pl.ds(NPAD + c0, CB)], dpb_v)
            for k in range(5):
                pltpu.sync_copy(x_hbm.at[k, pl.ds(c0, CB)],
                                xc_v.at[pl.ds(k * CB, CB)])

            def dloop(gi, carry2):
                o = gi * 16
                deg = dpa_v[pl.ds(o, 16)] + dpb_v[pl.ds(o, 16)] + 1.0
                dinv_v[pl.ds(o, 16)] = _rsqrt16(deg)
                return carry2

            lax.fori_loop(0, CB // 16, dloop, 0)

            def hloop(u, carry2):
                base = u * 2
                dv = plsc.load_gather(dinv_v, [d8 + base])
                acc = plsc.load_gather(xc_v, [d8 + base]) * w1vec[0]
                for k in range(1, 5):
                    acc = acc + plsc.load_gather(
                        xc_v, [d8 + (k * CB + base)]) * w1vec[k]
                plsc.store_scatter(rows_v, [d8 + base, i7], acc * dv)
                return carry2

            lax.fori_loop(0, CB // 2, hloop, 0)
            pltpu.sync_copy(rows_v, g_hbm.at[pl.ds(c0, CB)])
            pltpu.sync_copy(dinv_v, dinv_hbm.at[pl.ds(c0, CB)])
            return carry

        lax.fori_loop(0, NCB, chunk, 0)

    return kfn(dp, xt8, w1rep)


def _edge_pass(src3d, dst3d, g_rows, zeros8):
    @functools.partial(
        pl.kernel,
        mesh=_mesh,
        compiler_params=_sc_params,
        out_type=jax.ShapeDtypeStruct((NC * NPAD, F), jnp.float32),
        scratch_types=[
            pltpu.VMEM((K8, 8, 128), jnp.int32),
            pltpu.VMEM((K8, 8, 128), jnp.int32),
            pltpu.VMEM((2, 8, 128, F), jnp.float32),
            pltpu.VMEM_SHARED((NPAD, F), jnp.float32),
            pltpu.VMEM_SHARED((NPAD, F), jnp.float32),
            pltpu.SemaphoreType.DMA,
            pltpu.SemaphoreType.DMA,
        ],
    )
    def kfn(src_hbm, dst_hbm, g_hbm, z_hbm, out_hbm,
            src_v, dst_v, rows_v, g_sh, s_sh, gsem, ssem):
        c = lax.axis_index("c")
        s = lax.axis_index("s")
        wid = s * NC + c
        r0 = s * RPT
        pltpu.sync_copy(g_hbm.at[pl.ds(r0, RPT)], g_sh.at[pl.ds(r0, RPT)])
        pltpu.sync_copy(z_hbm.at[pl.ds(r0, RPT)], s_sh.at[pl.ds(r0, RPT)])
        plsc.subcore_barrier()
        vbase = wid * V8PW

        def chunk(i, carry):
            vb = vbase + i * K8
            pltpu.sync_copy(src_hbm.at[pl.ds(vb, K8)], src_v)
            pltpu.sync_copy(dst_hbm.at[pl.ds(vb, K8)], dst_v)

            def vec(j8, carry2):
                p = lax.rem(j8, 2)
                for r in range(8):
                    pltpu.async_copy(g_sh.at[src_v.at[j8, r]],
                                     rows_v.at[p, r], gsem)

                @pl.when(j8 > 0)
                def _():
                    for r in range(8):
                        pltpu.make_async_copy(
                            rows_v.at[1 - p, r],
                            s_sh.at[dst_v.at[j8 - 1, r]], ssem).wait()

                for r in range(8):
                    pltpu.make_async_copy(g_sh.at[src_v.at[j8, r]],
                                          rows_v.at[p, r], gsem).wait()
                    pltpu.async_copy(rows_v.at[p, r],
                                     s_sh.at[dst_v.at[j8, r]], ssem, add=True)
                return carry2

            lax.fori_loop(0, K8, vec, 0)
            for r in range(8):
                pltpu.make_async_copy(rows_v.at[(K8 - 1) % 2, r],
                                      s_sh.at[dst_v.at[K8 - 1, r]],
                                      ssem).wait()
            return carry

        lax.fori_loop(0, NCHUNK, chunk, 0)

        @pl.when(wid < V8 - REM8)
        def _():
            pltpu.sync_copy(src_hbm.at[pl.ds(REM8 + wid, 1)],
                            src_v.at[pl.ds(0, 1)])
            pltpu.sync_copy(dst_hbm.at[pl.ds(REM8 + wid, 1)],
                            dst_v.at[pl.ds(0, 1)])
            for r in range(8):
                pltpu.sync_copy(g_sh.at[src_v.at[0, r]], rows_v.at[0, r])
                pltpu.sync_copy(rows_v.at[0, r], s_sh.at[dst_v.at[0, r]],
                                add=True)

        plsc.subcore_barrier()
        pltpu.sync_copy(s_sh.at[pl.ds(r0, RPT)],
                        out_hbm.at[pl.ds(c * NPAD + r0, RPT)])

    return kfn(src3d, dst3d, g_rows, zeros8)


def _stage2(s1, g1, dinv, w2rep, b1rep):
    @functools.partial(
        pl.kernel,
        mesh=_mesh,
        compiler_params=_sc_params,
        out_type=jax.ShapeDtypeStruct((NPAD, F), jnp.float32),
        scratch_types=[
            pltpu.VMEM((CB, F), jnp.float32),
            pltpu.VMEM((CB, F), jnp.float32),
            pltpu.VMEM((CB, F), jnp.float32),
            pltpu.VMEM((CB,), jnp.float32),
            pltpu.VMEM((CB, F), jnp.float32),
            pltpu.VMEM((CB, F), jnp.float32),
            pltpu.VMEM((8, 16), jnp.float32),
            pltpu.VMEM((16,), jnp.float32),
        ],
    )
    def kfn(s_hbm, g_hbm, dinv_hbm, w2_hbm, b1_hbm, out_hbm,
            sa_v, sb_v, gc_v, dc_v, t_v, o_v, w2_v, b1_v):
        c = lax.axis_index("c")
        s = lax.axis_index("s")
        wid = s * NC + c
        n0 = wid * NPT
        pltpu.sync_copy(w2_hbm, w2_v)
        pltpu.sync_copy(b1_hbm, b1_v)
        d8, i7 = _lane_d8()
        w2vec = [w2_v[k, :] for k in range(8)]
        b1vec = b1_v[...]

        def chunk(ci, carry):
            c0 = n0 + ci * CB
            pltpu.sync_copy(s_hbm.at[pl.ds(c0, CB)], sa_v)
            pltpu.sync_copy(s_hbm.at[pl.ds(NPAD + c0, CB)], sb_v)
            pltpu.sync_copy(g_hbm.at[pl.ds(c0, CB)], gc_v)
            pltpu.sync_copy(dinv_hbm.at[pl.ds(c0, CB)], dc_v)

            def tloop(u, carry2):
                rw = d8 + u * 2
                dv = plsc.load_gather(dc_v, [rw])
                tt = plsc.load_gather(sa_v, [rw, i7]) \
                    + plsc.load_gather(sb_v, [rw, i7]) \
                    + plsc.load_gather(gc_v, [rw, i7])
                plsc.store_scatter(t_v, [rw, i7], dv * tt + b1vec)
                return carry2

            lax.fori_loop(0, CB // 2, tloop, 0)

            def hloop(u, carry2):
                rw = d8 + u * 2
                dv = plsc.load_gather(dc_v, [rw])
                acc = plsc.load_gather(
                    t_v, [rw, jnp.zeros((16,), jnp.int32)]) * w2vec[0]
                for k in range(1, 8):
                    acc = acc + plsc.load_gather(
                        t_v, [rw, jnp.full((16,), k, jnp.int32)]) * w2vec[k]
                plsc.store_scatter(o_v, [rw, i7], acc * dv)
                return carry2

            lax.fori_loop(0, CB // 2, hloop, 0)
            pltpu.sync_copy(o_v, out_hbm.at[pl.ds(c0, CB)])
            return carry

        lax.fori_loop(0, NCB, chunk, 0)

    return kfn(s1, g1, dinv, w2rep, b1rep)


def _stage3(s2, g2, dinv, b2rep):
    @functools.partial(
        pl.kernel,
        mesh=_mesh,
        compiler_params=_sc_params,
        out_type=jax.ShapeDtypeStruct((NPAD, F), jnp.float32),
        scratch_types=[
            pltpu.VMEM((CB, F), jnp.float32),
            pltpu.VMEM((CB, F), jnp.float32),
            pltpu.VMEM((CB, F), jnp.float32),
            pltpu.VMEM((CB,), jnp.float32),
            pltpu.VMEM((CB, F), jnp.float32),
            pltpu.VMEM((16,), jnp.float32),
        ],
    )
    def kfn(s_hbm, g_hbm, dinv_hbm, b2_hbm, out_hbm,
            sa_v, sb_v, gc_v, dc_v, o_v, b2_v):
        c = lax.axis_index("c")
        s = lax.axis_index("s")
        wid = s * NC + c
        n0 = wid * NPT
        pltpu.sync_copy(b2_hbm, b2_v)
        d8, i7 = _lane_d8()
        b2vec = b2_v[...]

        def chunk(ci, carry):
            c0 = n0 + ci * CB
            pltpu.sync_copy(s_hbm.at[pl.ds(c0, CB)], sa_v)
            pltpu.sync_copy(s_hbm.at[pl.ds(NPAD + c0, CB)], sb_v)
            pltpu.sync_copy(g_hbm.at[pl.ds(c0, CB)], gc_v)
            pltpu.sync_copy(dinv_hbm.at[pl.ds(c0, CB)], dc_v)

            def oloop(u, carry2):
                rw = d8 + u * 2
                dv = plsc.load_gather(dc_v, [rw])
                tt = plsc.load_gather(sa_v, [rw, i7]) \
                    + plsc.load_gather(sb_v, [rw, i7]) \
                    + plsc.load_gather(gc_v, [rw, i7])
                plsc.store_scatter(o_v, [rw, i7], dv * tt + b2vec)
                return carry2

            lax.fori_loop(0, CB // 2, oloop, 0)
            pltpu.sync_copy(o_v, out_hbm.at[pl.ds(c0, CB)])
            return carry

        lax.fori_loop(0, NCB, chunk, 0)

    return kfn(s2, g2, dinv, b2rep)


def kernel(x, edge_index, edge_f, edge_attr, W1, b1, W2, b2):
    src3d = edge_index[0].reshape(V8, 8, 128)
    dst3d = edge_index[1].reshape(V8, 8, 128)
    zeros1 = jnp.zeros((NPAD,), jnp.float32)
    zeros8 = jnp.zeros((NPAD, F), jnp.float32)
    xt8 = jnp.pad(x.T, ((0, 3), (0, NPAD - N)))
    w1rep = jnp.tile(W1, (1, 2))
    w2rep = jnp.tile(jnp.pad(W2, ((0, 0), (0, 3))), (1, 2))
    b1rep = jnp.tile(b1, 2)
    b2rep = jnp.tile(jnp.pad(b2, (0, 3)), 2)

    dp = _deg_pass(dst3d, zeros1)
    g1, dinv = _stage1(dp, xt8, w1rep)
    s1 = _edge_pass(src3d, dst3d, g1, zeros8)
    g2 = _stage2(s1, g1, dinv, w2rep, b1rep)
    s2 = _edge_pass(src3d, dst3d, g2, zeros8)
    outf = _stage3(s2, g2, dinv, b2rep)
    return outf[:N, :5]


# speedup vs baseline: 208.3260x; 1.1531x over previous
import functools

import jax
import jax.numpy as jnp
from jax import lax
from jax.experimental import pallas as pl
from jax.experimental.pallas import tpu as pltpu
from jax.experimental.pallas import tpu_sc as plsc

N = 100000
E = 6400000
NPAD = 100352
F = 8
NC, NS = 2, 16
NW = NC * NS
RPT = NPAD // NS
V8 = E // 1024
V8PW = V8 // NW
K8 = 3
NCHUNK = V8PW // K8
K8D = 13
NCHD = V8PW // K8D
REM8 = NW * V8PW
NPT = NPAD // NW
CB = 448
NCB = NPT // CB
CBR = CB * F // 16

_mesh = plsc.VectorSubcoreMesh(core_axis_name="c", subcore_axis_name="s")
_sc_params = pltpu.CompilerParams(use_tc_tiling_on_sc=False,
                                 needs_layout_passes=False)


def _lane_d8():
    iota = lax.iota(jnp.int32, 16)
    return (lax.shift_right_logical(iota, 3),
            lax.bitwise_and(iota, 7))


def _rsqrt16(deg):
    i = plsc.bitcast(deg, jnp.int32)
    i = jnp.int32(0x5F3759DF) - lax.shift_right_arithmetic(i, 1)
    y = plsc.bitcast(i, jnp.float32)
    for _ in range(3):
        y = y * (1.5 - 0.5 * deg * y * y)
    return y


def _deg_pass(dst3d, zeros1):
    @functools.partial(
        pl.kernel,
        mesh=_mesh,
        compiler_params=_sc_params,
        out_type=jax.ShapeDtypeStruct((NC * NPAD,), jnp.float32),
        scratch_types=[
            pltpu.VMEM((2, K8D, 8, 128), jnp.int32),
            pltpu.VMEM((128,), jnp.float32),
            pltpu.VMEM_SHARED((NPAD,), jnp.float32),
            pltpu.SemaphoreType.DMA,
            pltpu.SemaphoreType.DMA,
        ],
    )
    def kfn(dst_hbm, z_hbm, out_hbm, dst_v, ones_v, deg_sh, ssem, isem):
        c = lax.axis_index("c")
        s = lax.axis_index("s")
        wid = s * NC + c
        for i in range(8):
            ones_v[pl.ds(i * 16, 16)] = jnp.ones((16,), jnp.float32)
        r0 = s * RPT
        pltpu.sync_copy(z_hbm.at[pl.ds(r0, RPT)], deg_sh.at[pl.ds(r0, RPT)])
        plsc.subcore_barrier()
        vbase = wid * V8PW
        pltpu.async_copy(dst_hbm.at[pl.ds(vbase, K8D)], dst_v.at[0], isem)

        def chunk(i, carry):
            di = lax.rem(i, 2)

            @pl.when(i + 1 < NCHD)
            def _():
                pltpu.async_copy(
                    dst_hbm.at[pl.ds(vbase + (i + 1) * K8D, K8D)],
                    dst_v.at[1 - di], isem)

            pltpu.make_async_copy(dst_hbm.at[pl.ds(vbase + i * K8D, K8D)],
                                  dst_v.at[di], isem).wait()

            def vec(j8, carry2):
                for r in range(8):
                    pltpu.async_copy(ones_v, deg_sh.at[dst_v.at[di, j8, r]],
                                     ssem, add=True)

                @pl.when(j8 > 0)
                def _():
                    for r in range(8):
                        pltpu.make_async_copy(
                            ones_v, deg_sh.at[dst_v.at[di, j8 - 1, r]], ssem
                        ).wait()

                return carry2

            lax.fori_loop(0, K8D, vec, 0)
            for r in range(8):
                pltpu.make_async_copy(
                    ones_v, deg_sh.at[dst_v.at[di, K8D - 1, r]], ssem).wait()
            return carry

        lax.fori_loop(0, NCHD, chunk, 0)

        @pl.when(wid < V8 - REM8)
        def _():
            pltpu.sync_copy(dst_hbm.at[pl.ds(REM8 + wid, 1)],
                            dst_v.at[0, pl.ds(0, 1)])
            for r in range(8):
                pltpu.sync_copy(ones_v, deg_sh.at[dst_v.at[0, 0, r]],
                                add=True)

        plsc.subcore_barrier()
        pltpu.sync_copy(deg_sh.at[pl.ds(r0, RPT)],
                        out_hbm.at[pl.ds(c * NPAD + r0, RPT)])

    return kfn(dst3d, zeros1)


def _stage1(dp, xt8, w1rep):
    @functools.partial(
        pl.kernel,
        mesh=_mesh,
        compiler_params=_sc_params,
        out_type=[jax.ShapeDtypeStruct((NPAD, F), jnp.float32),
                  jax.ShapeDtypeStruct((NPAD,), jnp.float32)],
        scratch_types=[
            pltpu.VMEM((CB,), jnp.float32),
            pltpu.VMEM((CB,), jnp.float32),
            pltpu.VMEM((5 * CB,), jnp.float32),
            pltpu.VMEM((CB,), jnp.float32),
            pltpu.VMEM((CB, F), jnp.float32),
            pltpu.VMEM((5, 16), jnp.float32),
        ],
    )
    def kfn(dp_hbm, x_hbm, w1_hbm, g_hbm, dinv_hbm,
            dpa_v, dpb_v, xc_v, dinv_v, rows_v, w1_v):
        c = lax.axis_index("c")
        s = lax.axis_index("s")
        wid = s * NC + c
        n0 = wid * NPT
        pltpu.sync_copy(w1_hbm, w1_v)
        d8, i7 = _lane_d8()
        w1vec = [w1_v[k, :] for k in range(5)]

        def chunk(ci, carry):
            c0 = n0 + ci * CB
            pltpu.sync_copy(dp_hbm.at[pl.ds(c0, CB)], dpa_v)
            pltpu.sync_copy(dp_hbm.at[pl.ds(NPAD + c0, CB)], dpb_v)
            for k in range(5):
                pltpu.sync_copy(x_hbm.at[k, pl.ds(c0, CB)],
                                xc_v.at[pl.ds(k * CB, CB)])

            def dloop(gi, carry2):
                o = gi * 16
                deg = dpa_v[pl.ds(o, 16)] + dpb_v[pl.ds(o, 16)] + 1.0
                dinv_v[pl.ds(o, 16)] = _rsqrt16(deg)
                return carry2

            lax.fori_loop(0, CB // 16, dloop, 0)

            def hloop(u, carry2):
                base = u * 2
                dv = plsc.load_gather(dinv_v, [d8 + base])
                acc = plsc.load_gather(xc_v, [d8 + base]) * w1vec[0]
                for k in range(1, 5):
                    acc = acc + plsc.load_gather(
                        xc_v, [d8 + (k * CB + base)]) * w1vec[k]
                plsc.store_scatter(rows_v, [d8 + base, i7], acc * dv)
                return carry2

            lax.fori_loop(0, CB // 2, hloop, 0)
            pltpu.sync_copy(rows_v, g_hbm.at[pl.ds(c0, CB)])
            pltpu.sync_copy(dinv_v, dinv_hbm.at[pl.ds(c0, CB)])
            return carry

        lax.fori_loop(0, NCB, chunk, 0)

    return kfn(dp, xt8, w1rep)


def _edge_pass(src3d, dst3d, g_rows, zeros8):
    @functools.partial(
        pl.kernel,
        mesh=_mesh,
        compiler_params=_sc_params,
        out_type=jax.ShapeDtypeStruct((NC * NPAD, F), jnp.float32),
        scratch_types=[
            pltpu.VMEM((2, K8, 8, 128), jnp.int32),
            pltpu.VMEM((2, K8, 8, 128), jnp.int32),
            pltpu.VMEM((2, 8, 128, F), jnp.float32),
            pltpu.VMEM_SHARED((NPAD, F), jnp.float32),
            pltpu.VMEM_SHARED((NPAD, F), jnp.float32),
            pltpu.SemaphoreType.DMA,
            pltpu.SemaphoreType.DMA,
            pltpu.SemaphoreType.DMA,
        ],
    )
    def kfn(src_hbm, dst_hbm, g_hbm, z_hbm, out_hbm,
            src_v, dst_v, rows_v, g_sh, s_sh, gsem, ssem, isem):
        c = lax.axis_index("c")
        s = lax.axis_index("s")
        wid = s * NC + c
        r0 = s * RPT
        pltpu.sync_copy(g_hbm.at[pl.ds(r0, RPT)], g_sh.at[pl.ds(r0, RPT)])
        pltpu.sync_copy(z_hbm.at[pl.ds(r0, RPT)], s_sh.at[pl.ds(r0, RPT)])
        plsc.subcore_barrier()
        vbase = wid * V8PW
        pltpu.async_copy(src_hbm.at[pl.ds(vbase, K8)], src_v.at[0], isem)
        pltpu.async_copy(dst_hbm.at[pl.ds(vbase, K8)], dst_v.at[0], isem)

        def chunk(i, carry):
            di = lax.rem(i, 2)
            vb = vbase + i * K8

            @pl.when(i + 1 < NCHUNK)
            def _():
                pltpu.async_copy(src_hbm.at[pl.ds(vb + K8, K8)],
                                 src_v.at[1 - di], isem)
                pltpu.async_copy(dst_hbm.at[pl.ds(vb + K8, K8)],
                                 dst_v.at[1 - di], isem)

            pltpu.make_async_copy(src_hbm.at[pl.ds(vb, K8)],
                                  src_v.at[di], isem).wait()
            pltpu.make_async_copy(dst_hbm.at[pl.ds(vb, K8)],
                                  dst_v.at[di], isem).wait()

            def vec(j8, carry2):
                p = lax.rem(j8, 2)
                for r in range(8):
                    pltpu.async_copy(g_sh.at[src_v.at[di, j8, r]],
                                     rows_v.at[p, r], gsem)

                @pl.when(j8 > 0)
                def _():
                    for r in range(8):
                        pltpu.make_async_copy(
                            rows_v.at[1 - p, r],
                            s_sh.at[dst_v.at[di, j8 - 1, r]], ssem).wait()

                for r in range(8):
                    pltpu.make_async_copy(g_sh.at[src_v.at[di, j8, r]],
                                          rows_v.at[p, r], gsem).wait()
                    pltpu.async_copy(rows_v.at[p, r],
                                     s_sh.at[dst_v.at[di, j8, r]], ssem,
                                     add=True)
                return carry2

            lax.fori_loop(0, K8, vec, 0)
            for r in range(8):
                pltpu.make_async_copy(rows_v.at[(K8 - 1) % 2, r],
                                      s_sh.at[dst_v.at[di, K8 - 1, r]],
                                      ssem).wait()
            return carry

        lax.fori_loop(0, NCHUNK, chunk, 0)

        @pl.when(wid < V8 - REM8)
        def _():
            pltpu.sync_copy(src_hbm.at[pl.ds(REM8 + wid, 1)],
                            src_v.at[0, pl.ds(0, 1)])
            pltpu.sync_copy(dst_hbm.at[pl.ds(REM8 + wid, 1)],
                            dst_v.at[0, pl.ds(0, 1)])
            for r in range(8):
                pltpu.sync_copy(g_sh.at[src_v.at[0, 0, r]], rows_v.at[0, r])
                pltpu.sync_copy(rows_v.at[0, r], s_sh.at[dst_v.at[0, 0, r]],
                                add=True)

        plsc.subcore_barrier()
        pltpu.sync_copy(s_sh.at[pl.ds(r0, RPT)],
                        out_hbm.at[pl.ds(c * NPAD + r0, RPT)])

    return kfn(src3d, dst3d, g_rows, zeros8)


def _stage2(s1, g1, dinv, w2rep, b1rep):
    @functools.partial(
        pl.kernel,
        mesh=_mesh,
        compiler_params=_sc_params,
        out_type=jax.ShapeDtypeStruct((NPAD, F), jnp.float32),
        scratch_types=[
            pltpu.VMEM((CB, F), jnp.float32),
            pltpu.VMEM((CB, F), jnp.float32),
            pltpu.VMEM((CB, F), jnp.float32),
            pltpu.VMEM((CB,), jnp.float32),
            pltpu.VMEM((CB, F), jnp.float32),
            pltpu.VMEM((CB, F), jnp.float32),
            pltpu.VMEM((8, 16), jnp.float32),
            pltpu.VMEM((16,), jnp.float32),
        ],
    )
    def kfn(s_hbm, g_hbm, dinv_hbm, w2_hbm, b1_hbm, out_hbm,
            sa_v, sb_v, gc_v, dc_v, t_v, o_v, w2_v, b1_v):
        c = lax.axis_index("c")
        s = lax.axis_index("s")
        wid = s * NC + c
        n0 = wid * NPT
        pltpu.sync_copy(w2_hbm, w2_v)
        pltpu.sync_copy(b1_hbm, b1_v)
        d8, i7 = _lane_d8()
        w2vec = [w2_v[k, :] for k in range(8)]
        b1vec = b1_v[...]

        def chunk(ci, carry):
            c0 = n0 + ci * CB
            pltpu.sync_copy(s_hbm.at[pl.ds(c0, CB)], sa_v)
            pltpu.sync_copy(s_hbm.at[pl.ds(NPAD + c0, CB)], sb_v)
            pltpu.sync_copy(g_hbm.at[pl.ds(c0, CB)], gc_v)
            pltpu.sync_copy(dinv_hbm.at[pl.ds(c0, CB)], dc_v)

            def tloop(u, carry2):
                rw = d8 + u * 2
                dv = plsc.load_gather(dc_v, [rw])
                tt = plsc.load_gather(sa_v, [rw, i7]) \
                    + plsc.load_gather(sb_v, [rw, i7]) \
                    + plsc.load_gather(gc_v, [rw, i7])
                plsc.store_scatter(t_v, [rw, i7], dv * tt + b1vec)
                return carry2

            lax.fori_loop(0, CB // 2, tloop, 0)

            def hloop(u, carry2):
                rw = d8 + u * 2
                dv = plsc.load_gather(dc_v, [rw])
                acc = plsc.load_gather(
                    t_v, [rw, jnp.zeros((16,), jnp.int32)]) * w2vec[0]
                for k in range(1, 8):
                    acc = acc + plsc.load_gather(
                        t_v, [rw, jnp.full((16,), k, jnp.int32)]) * w2vec[k]
                plsc.store_scatter(o_v, [rw, i7], acc * dv)
                return carry2

            lax.fori_loop(0, CB // 2, hloop, 0)
            pltpu.sync_copy(o_v, out_hbm.at[pl.ds(c0, CB)])
            return carry

        lax.fori_loop(0, NCB, chunk, 0)

    return kfn(s1, g1, dinv, w2rep, b1rep)


def _stage3(s2, g2, dinv, b2rep):
    @functools.partial(
        pl.kernel,
        mesh=_mesh,
        compiler_params=_sc_params,
        out_type=jax.ShapeDtypeStruct((NPAD, F), jnp.float32),
        scratch_types=[
            pltpu.VMEM((CB, F), jnp.float32),
            pltpu.VMEM((CB, F), jnp.float32),
            pltpu.VMEM((CB, F), jnp.float32),
            pltpu.VMEM((CB,), jnp.float32),
            pltpu.VMEM((CB, F), jnp.float32),
            pltpu.VMEM((16,), jnp.float32),
        ],
    )
    def kfn(s_hbm, g_hbm, dinv_hbm, b2_hbm, out_hbm,
            sa_v, sb_v, gc_v, dc_v, o_v, b2_v):
        c = lax.axis_index("c")
        s = lax.axis_index("s")
        wid = s * NC + c
        n0 = wid * NPT
        pltpu.sync_copy(b2_hbm, b2_v)
        d8, i7 = _lane_d8()
        b2vec = b2_v[...]

        def chunk(ci, carry):
            c0 = n0 + ci * CB
            pltpu.sync_copy(s_hbm.at[pl.ds(c0, CB)], sa_v)
            pltpu.sync_copy(s_hbm.at[pl.ds(NPAD + c0, CB)], sb_v)
            pltpu.sync_copy(g_hbm.at[pl.ds(c0, CB)], gc_v)
            pltpu.sync_copy(dinv_hbm.at[pl.ds(c0, CB)], dc_v)

            def oloop(u, carry2):
                rw = d8 + u * 2
                dv = plsc.load_gather(dc_v, [rw])
                tt = plsc.load_gather(sa_v, [rw, i7]) \
                    + plsc.load_gather(sb_v, [rw, i7]) \
                    + plsc.load_gather(gc_v, [rw, i7])
                plsc.store_scatter(o_v, [rw, i7], dv * tt + b2vec)
                return carry2

            lax.fori_loop(0, CB // 2, oloop, 0)
            pltpu.sync_copy(o_v, out_hbm.at[pl.ds(c0, CB)])
            return carry

        lax.fori_loop(0, NCB, chunk, 0)

    return kfn(s2, g2, dinv, b2rep)


def kernel(x, edge_index, edge_f, edge_attr, W1, b1, W2, b2):
    src3d = edge_index[0].reshape(V8, 8, 128)
    dst3d = edge_index[1].reshape(V8, 8, 128)
    zeros1 = jnp.zeros((NPAD,), jnp.float32)
    zeros8 = jnp.zeros((NPAD, F), jnp.float32)
    xt8 = jnp.pad(x.T, ((0, 3), (0, NPAD - N)))
    w1rep = jnp.tile(W1, (1, 2))
    w2rep = jnp.tile(jnp.pad(W2, ((0, 0), (0, 3))), (1, 2))
    b1rep = jnp.tile(b1, 2)
    b2rep = jnp.tile(jnp.pad(b2, (0, 3)), 2)

    dp = _deg_pass(dst3d, zeros1)
    g1, dinv = _stage1(dp, xt8, w1rep)
    s1 = _edge_pass(src3d, dst3d, g1, zeros8)
    g2 = _stage2(s1, g1, dinv, w2rep, b1rep)
    s2 = _edge_pass(src3d, dst3d, g2, zeros8)
    outf = _stage3(s2, g2, dinv, b2rep)
    return outf[:N, :5]
